# baseline (speedup 1.0000x reference)
import functools

import jax
import jax.numpy as jnp
from jax import lax
from jax.experimental import pallas as pl
from jax.experimental.pallas import tpu as pltpu
from jax.experimental.pallas import tpu_sc as plsc

NC = 2
NS = 16
NW = NC * NS

COORDS_RANGE_OVER_NORM = 15.0 / 100.0


def _ab_body(h_ref, wa_ref, wb_ref, a_ref, b_ref):
    hb = h_ref[...].astype(jnp.bfloat16)
    dn = (((1,), (1,)), ((), ()))
    a_ref[...] = lax.dot_general(
        hb, wa_ref[...], dn, preferred_element_type=jnp.float32)
    b_ref[...] = lax.dot_general(
        hb, wb_ref[...], dn, preferred_element_type=jnp.float32)


def _node_ab(h, w1a, w1b):
    n, hdim = h.shape
    return pl.pallas_call(
        _ab_body,
        out_shape=(
            jax.ShapeDtypeStruct((n, hdim), jnp.float32),
            jax.ShapeDtypeStruct((n, hdim), jnp.float32),
        ),
    )(h, w1a, w1b)


IW = 80
TPC = 4
CG = IW * TPC
NSLOT = 3


def _gather_sum(a_t, b_t, row1d, col1d, e0, epw, gpad):
    sizes = [CG] * (epw // CG)
    if epw % CG:
        sizes.append(epw % CG)
    mesh = plsc.VectorSubcoreMesh(core_axis_name="c", subcore_axis_name="s")
    hdim = a_t.shape[1]

    @functools.partial(
        pl.kernel,
        out_type=jax.ShapeDtypeStruct((gpad, hdim), jnp.float32),
        mesh=mesh,
        scratch_types=[
            pltpu.VMEM((NSLOT * CG,), jnp.int32),
            pltpu.VMEM((NSLOT * CG,), jnp.int32),
            pltpu.VMEM((NSLOT * CG, hdim), jnp.float32),
            pltpu.SemaphoreType.DMA,
            pltpu.SemaphoreType.DMA,
            pltpu.SemaphoreType.DMA,
        ],
    )
    def k(a_hbm, b_hbm, row_hbm, col_hbm, g_hbm, rowv, colv, buf,
          sema, semb, semw):
        wid = lax.axis_index("c") * NS + lax.axis_index("s")
        lbase0 = wid * epw
        n = len(sizes)
        offs = [0]
        for sz in sizes:
            offs.append(offs[-1] + sz)

        def load_idx(c):
            so = (c % NSLOT) * CG
            sz = sizes[c]
            lbase = lbase0 + offs[c]
            pltpu.sync_copy(row_hbm.at[pl.ds(e0 + lbase, sz)],
                            rowv.at[pl.ds(so, sz)])
            pltpu.sync_copy(col_hbm.at[pl.ds(e0 + lbase, sz)],
                            colv.at[pl.ds(so, sz)])

        def gather(c, idxv, tbl, sem, add):
            so = (c % NSLOT) * CG
            return [
                pltpu.async_copy(
                    tbl.at[idxv.at[pl.ds(so + j * IW, IW)]],
                    buf.at[pl.ds(so + j * IW, IW)], sem, add=add)
                for j in range(sizes[c] // IW)
            ]

        wb = {}
        load_idx(0)
        ga = gather(0, rowv, a_hbm, sema, False)
        for c in range(n):
            for d in ga:
                d.wait()
            gb = gather(c, colv, b_hbm, semb, True)
            if c + 1 < n:
                if c + 1 - NSLOT >= 0:
                    wb.pop(c + 1 - NSLOT).wait()
                load_idx(c + 1)
                ga = gather(c + 1, rowv, a_hbm, sema, False)
            for d in gb:
                d.wait()
            so = (c % NSLOT) * CG
            wb[c] = pltpu.async_copy(
                buf.at[pl.ds(so, sizes[c])],
                g_hbm.at[pl.ds(lbase0 + offs[c], sizes[c])], semw)
        for c in sorted(wb):
            wb[c].wait()

    return k(a_t, b_t, row1d, col1d)


BE = 8192
EPAD = 327680


def _silu(x):
    h = x * jnp.bfloat16(0.5)
    return h + h * jnp.tanh(h)


def _mlp_body(g_ref, ea_ref, w1c_ref, b1_ref, w2_ref, b2_ref, w3_ref, t_ref):
    dn = (((1,), (1,)), ((), ()))
    pre = g_ref[...] + lax.dot_general(
        ea_ref[...].astype(jnp.bfloat16), w1c_ref[...],
        (((0,), (1,)), ((), ())), preferred_element_type=jnp.float32)
    pre = (pre + b1_ref[...]).astype(jnp.bfloat16)
    x = _silu(pre)
    pre2 = (lax.dot_general(
        x, w2_ref[...], dn, preferred_element_type=jnp.float32)
        + b2_ref[...]).astype(jnp.bfloat16)
    y = _silu(pre2)
    s = lax.dot_general(
        w3_ref[...], y, dn, preferred_element_type=jnp.float32)
    t_ref[...] = (jnp.tanh(s) * COORDS_RANGE_OVER_NORM)[0]


def _edge_mlp(g, ea_t, blk0, w1c, b1r, w2, b2r, w3):
    gpad, hdim = g.shape
    de = ea_t.shape[0]
    grid = (gpad // BE,)

    return pl.pallas_call(
        _mlp_body,
        grid=grid,
        in_specs=[
            pl.BlockSpec((BE, hdim), lambda i: (i, 0)),
            pl.BlockSpec((de, BE), lambda i: (0, blk0 + i)),
            pl.BlockSpec((hdim, de), lambda i: (0, 0)),
            pl.BlockSpec((1, hdim), lambda i: (0, 0)),
            pl.BlockSpec((hdim, hdim), lambda i: (0, 0)),
            pl.BlockSpec((1, hdim), lambda i: (0, 0)),
            pl.BlockSpec((1, hdim), lambda i: (0, 0)),
        ],
        out_specs=pl.BlockSpec((BE,), lambda i: (i,)),
        out_shape=jax.ShapeDtypeStruct((gpad,), jnp.float32),
    )(g, ea_t, w1c, b1r, w2, b2r, w3)


ACC = 32768
C4 = 2000


def _segment_scatter(t_loc, cd0, cd1, cd2, row1d, zeros1d, e0, epw):
    sizes = [C4] * (epw // C4)
    if epw % C4:
        sizes.append(epw % C4)
    mesh = plsc.VectorSubcoreMesh(core_axis_name="c", subcore_axis_name="s")
    sl = ACC // NS

    @functools.partial(
        pl.kernel,
        out_type=jax.ShapeDtypeStruct((NC, ACC // 128, 128), jnp.float32),
        mesh=mesh,
        scratch_types=[
            pltpu.VMEM((C4,), jnp.int32),
            pltpu.VMEM((C4,), jnp.int32),
            pltpu.VMEM((C4,), jnp.float32),
            pltpu.VMEM((C4,), jnp.float32),
            pltpu.VMEM((C4,), jnp.float32),
            pltpu.VMEM((C4,), jnp.float32),
            pltpu.VMEM((C4,), jnp.float32),
            pltpu.VMEM((C4,), jnp.float32),
            pltpu.VMEM((C4,), jnp.float32),
            pltpu.VMEM((C4,), jnp.float32),
            pltpu.VMEM((ACC,), jnp.float32),
            pltpu.VMEM((NS, sl), jnp.float32),
            pltpu.VMEM((sl // 128, 128), jnp.float32),
            pltpu.VMEM_SHARED((NS, ACC), jnp.float32),
        ],
        compiler_params=pltpu.CompilerParams(needs_layout_passes=False),
    )
    def k(t_hbm, cd0_hbm, cd1_hbm, cd2_hbm, row_hbm, zero_hbm, out_hbm,
          rowv0, rowv1, tv0, tv1, c0v0, c0v1, c1v0, c1v1, c2v0, c2v1,
          accl, buf2, res, stage):
        cid = lax.axis_index("c")
        sid = lax.axis_index("s")
        wid = cid * NS + sid
        slots = [(rowv0, tv0, c0v0, c1v0, c2v0), (rowv1, tv1, c0v1, c1v1, c2v1)]

        pltpu.sync_copy(zero_hbm, accl)

        offs = [0]
        for sz in sizes:
            offs.append(offs[-1] + sz)

        def load(c, sem):
            rv, tvv, a0, a1, a2 = slots[c % 2]
            sz = sizes[c]
            lbase = wid * epw + offs[c]
            return [
                pltpu.async_copy(row_hbm.at[pl.ds(e0 + lbase, sz)],
                                 rv.at[pl.ds(0, sz)], sem),
                pltpu.async_copy(t_hbm.at[pl.ds(lbase, sz)],
                                 tvv.at[pl.ds(0, sz)], sem),
                pltpu.async_copy(cd0_hbm.at[pl.ds(e0 + lbase, sz)],
                                 a0.at[pl.ds(0, sz)], sem),
                pltpu.async_copy(cd1_hbm.at[pl.ds(e0 + lbase, sz)],
                                 a1.at[pl.ds(0, sz)], sem),
                pltpu.async_copy(cd2_hbm.at[pl.ds(e0 + lbase, sz)],
                                 a2.at[pl.ds(0, sz)], sem),
            ]

        def body(sem4):
            pend = load(0, sem4)
            for c, sz in enumerate(sizes):
                for d in pend:
                    d.wait()
                if c + 1 < len(sizes):
                    pend = load(c + 1, sem4)
                rv_b, tv_b, a0_b, a1_b, a2_b = slots[c % 2]

                def grp(g, c2):
                    rv = rv_b[pl.ds(g * 16, 16)]
                    th = tv_b[pl.ds(g * 16, 16)]
                    f0 = rv * 3
                    for d, cdv in enumerate((a0_b, a1_b, a2_b)):
                        cd_d = cdv[pl.ds(g * 16, 16)]
                        plsc.addupdate_scatter(accl, [f0 + d], cd_d * th)
                    return c2

                lax.fori_loop(0, sz // 16, grp, 0)

        pl.run_scoped(body, pltpu.SemaphoreType.DMA)

        pltpu.sync_copy(accl, stage.at[sid])
        plsc.subcore_barrier()
        pltpu.sync_copy(stage.at[:, pl.ds(sid * sl, sl)], buf2)

        def colsum_rows(jr, carry):
            for jc in range(8):
                j = jr * 8 + jc
                acc16 = buf2[0, pl.ds(j * 16, 16)]
                for r in range(1, NS):
                    acc16 = acc16 + buf2[r, pl.ds(j * 16, 16)]
                res[jr, pl.ds(jc * 16, 16)] = acc16
            return carry

        lax.fori_loop(0, sl // 128, colsum_rows, 0)
        pltpu.sync_copy(res, out_hbm.at[cid, pl.ds(sid * (sl // 128),
                                                   sl // 128)])

    return k(t_loc, cd0, cd1, cd2, row1d, zeros1d)


def kernel(h, coord, edge_index, coord_diff, edge_attr, W1, b1, W2, b2, W3):
    n, hdim = h.shape
    e = edge_index.shape[1]

    w1a = W1[:, :hdim].astype(jnp.bfloat16)
    w1b = W1[:, hdim:2 * hdim].astype(jnp.bfloat16)
    w1c = W1[:, 2 * hdim:].astype(jnp.bfloat16)

    a_t, b_t = _node_ab(h, w1a, w1b)

    row = edge_index[0]
    col = edge_index[1]

    h0 = 204800
    gpad_a = 204800
    gpad_b = 122880
    epw_a = h0 // NW
    epw_b = (e - h0) // NW

    ea_t = jnp.pad(edge_attr, ((0, EPAD - e), (0, 0))).T
    b1r = b1.reshape(1, -1)
    b2r = b2.reshape(1, -1)
    w2c = W2.astype(jnp.bfloat16)
    w3c = W3.astype(jnp.bfloat16)
    zeros1d = jnp.zeros((ACC,), dtype=jnp.float32)
    cd0, cd1, cd2 = coord_diff[:, 0], coord_diff[:, 1], coord_diff[:, 2]

    g_a = _gather_sum(a_t, b_t, row, col, 0, epw_a, gpad_a)
    g_b = _gather_sum(a_t, b_t, row, col, h0, epw_b, gpad_b)

    t_a = _edge_mlp(g_a, ea_t, 0, w1c, b1r, w2c, b2r, w3c)
    t_b = _edge_mlp(g_b, ea_t, h0 // BE, w1c, b1r, w2c, b2r, w3c)

    pa = _segment_scatter(t_a, cd0, cd1, cd2, row, zeros1d, 0, epw_a)
    pb = _segment_scatter(t_b, cd0, cd1, cd2, row, zeros1d, h0, epw_b)

    agg = ((pa[0] + pa[1]) + (pb[0] + pb[1])).reshape(-1)[:3 * n].reshape(n, 3)
    return coord + agg


# speedup vs baseline: 1.0024x; 1.0024x over previous
import functools

import jax
import jax.numpy as jnp
from jax import lax
from jax.experimental import pallas as pl
from jax.experimental.pallas import tpu as pltpu
from jax.experimental.pallas import tpu_sc as plsc

NC = 2
NS = 16
NW = NC * NS

COORDS_RANGE_OVER_NORM = 15.0 / 100.0


def _ab_body(h_ref, wa_ref, wb_ref, a_ref, b_ref):
    hb = h_ref[...].astype(jnp.bfloat16)
    dn = (((1,), (1,)), ((), ()))
    a_ref[...] = lax.dot_general(
        hb, wa_ref[...], dn, preferred_element_type=jnp.float32)
    b_ref[...] = lax.dot_general(
        hb, wb_ref[...], dn, preferred_element_type=jnp.float32)


def _node_ab(h, w1a, w1b):
    n, hdim = h.shape
    return pl.pallas_call(
        _ab_body,
        out_shape=(
            jax.ShapeDtypeStruct((n, hdim), jnp.float32),
            jax.ShapeDtypeStruct((n, hdim), jnp.float32),
        ),
    )(h, w1a, w1b)


IW = 80
TPC = 5
CG = IW * TPC
NSLOT = 2


def _gather_sum(a_t, b_t, row1d, col1d, e0, epw, gpad):
    sizes = [CG] * (epw // CG)
    if epw % CG:
        sizes.append(epw % CG)
    mesh = plsc.VectorSubcoreMesh(core_axis_name="c", subcore_axis_name="s")
    hdim = a_t.shape[1]

    @functools.partial(
        pl.kernel,
        out_type=jax.ShapeDtypeStruct((gpad, hdim), jnp.float32),
        mesh=mesh,
        scratch_types=[
            pltpu.VMEM((NSLOT * CG,), jnp.int32),
            pltpu.VMEM((NSLOT * CG,), jnp.int32),
            pltpu.VMEM((NSLOT * CG, hdim), jnp.float32),
            pltpu.SemaphoreType.DMA,
            pltpu.SemaphoreType.DMA,
            pltpu.SemaphoreType.DMA,
        ],
    )
    def k(a_hbm, b_hbm, row_hbm, col_hbm, g_hbm, rowv, colv, buf,
          sema, semb, semw):
        wid = lax.axis_index("c") * NS + lax.axis_index("s")
        lbase0 = wid * epw
        n = len(sizes)
        offs = [0]
        for sz in sizes:
            offs.append(offs[-1] + sz)

        def load_idx(c):
            so = (c % NSLOT) * CG
            sz = sizes[c]
            lbase = lbase0 + offs[c]
            pltpu.sync_copy(row_hbm.at[pl.ds(e0 + lbase, sz)],
                            rowv.at[pl.ds(so, sz)])
            pltpu.sync_copy(col_hbm.at[pl.ds(e0 + lbase, sz)],
                            colv.at[pl.ds(so, sz)])

        def gather(c, idxv, tbl, sem, add):
            so = (c % NSLOT) * CG
            return [
                pltpu.async_copy(
                    tbl.at[idxv.at[pl.ds(so + j * IW, IW)]],
                    buf.at[pl.ds(so + j * IW, IW)], sem, add=add)
                for j in range(sizes[c] // IW)
            ]

        wb = {}
        load_idx(0)
        ga = gather(0, rowv, a_hbm, sema, False)
        for c in range(n):
            for d in ga:
                d.wait()
            gb = gather(c, colv, b_hbm, semb, True)
            if c + 1 < n:
                if c + 1 - NSLOT >= 0:
                    wb.pop(c + 1 - NSLOT).wait()
                load_idx(c + 1)
                ga = gather(c + 1, rowv, a_hbm, sema, False)
            for d in gb:
                d.wait()
            so = (c % NSLOT) * CG
            wb[c] = pltpu.async_copy(
                buf.at[pl.ds(so, sizes[c])],
                g_hbm.at[pl.ds(lbase0 + offs[c], sizes[c])], semw)
        for c in sorted(wb):
            wb[c].wait()

    return k(a_t, b_t, row1d, col1d)


BE = 8192
EPAD = 327680


def _silu(x):
    h = x * jnp.bfloat16(0.5)
    return h + h * jnp.tanh(h)


def _mlp_body(g_ref, ea_ref, w1c_ref, b1_ref, w2_ref, b2_ref, w3_ref, t_ref):
    dn = (((1,), (1,)), ((), ()))
    pre = g_ref[...] + lax.dot_general(
        ea_ref[...].astype(jnp.bfloat16), w1c_ref[...],
        (((0,), (1,)), ((), ())), preferred_element_type=jnp.float32)
    pre = (pre + b1_ref[...]).astype(jnp.bfloat16)
    x = _silu(pre)
    pre2 = (lax.dot_general(
        x, w2_ref[...], dn, preferred_element_type=jnp.float32)
        + b2_ref[...]).astype(jnp.bfloat16)
    y = _silu(pre2)
    s = lax.dot_general(
        w3_ref[...], y, dn, preferred_element_type=jnp.float32)
    t_ref[...] = (jnp.tanh(s) * COORDS_RANGE_OVER_NORM)[0]


def _edge_mlp(g, ea_t, blk0, w1c, b1r, w2, b2r, w3):
    gpad, hdim = g.shape
    de = ea_t.shape[0]
    grid = (gpad // BE,)

    return pl.pallas_call(
        _mlp_body,
        grid=grid,
        in_specs=[
            pl.BlockSpec((BE, hdim), lambda i: (i, 0)),
            pl.BlockSpec((de, BE), lambda i: (0, blk0 + i)),
            pl.BlockSpec((hdim, de), lambda i: (0, 0)),
            pl.BlockSpec((1, hdim), lambda i: (0, 0)),
            pl.BlockSpec((hdim, hdim), lambda i: (0, 0)),
            pl.BlockSpec((1, hdim), lambda i: (0, 0)),
            pl.BlockSpec((1, hdim), lambda i: (0, 0)),
        ],
        out_specs=pl.BlockSpec((BE,), lambda i: (i,)),
        out_shape=jax.ShapeDtypeStruct((gpad,), jnp.float32),
    )(g, ea_t, w1c, b1r, w2, b2r, w3)


ACC = 32768
C4 = 2000


def _segment_scatter(t_loc, cd0, cd1, cd2, row1d, zeros1d, e0, epw):
    sizes = [C4] * (epw // C4)
    if epw % C4:
        sizes.append(epw % C4)
    mesh = plsc.VectorSubcoreMesh(core_axis_name="c", subcore_axis_name="s")
    sl = ACC // NS

    @functools.partial(
        pl.kernel,
        out_type=jax.ShapeDtypeStruct((NC, ACC // 128, 128), jnp.float32),
        mesh=mesh,
        scratch_types=[
            pltpu.VMEM((C4,), jnp.int32),
            pltpu.VMEM((C4,), jnp.int32),
            pltpu.VMEM((C4,), jnp.float32),
            pltpu.VMEM((C4,), jnp.float32),
            pltpu.VMEM((C4,), jnp.float32),
            pltpu.VMEM((C4,), jnp.float32),
            pltpu.VMEM((C4,), jnp.float32),
            pltpu.VMEM((C4,), jnp.float32),
            pltpu.VMEM((C4,), jnp.float32),
            pltpu.VMEM((C4,), jnp.float32),
            pltpu.VMEM((ACC,), jnp.float32),
            pltpu.VMEM((NS, sl), jnp.float32),
            pltpu.VMEM((sl // 128, 128), jnp.float32),
            pltpu.VMEM_SHARED((NS, ACC), jnp.float32),
        ],
        compiler_params=pltpu.CompilerParams(needs_layout_passes=False),
    )
    def k(t_hbm, cd0_hbm, cd1_hbm, cd2_hbm, row_hbm, zero_hbm, out_hbm,
          rowv0, rowv1, tv0, tv1, c0v0, c0v1, c1v0, c1v1, c2v0, c2v1,
          accl, buf2, res, stage):
        cid = lax.axis_index("c")
        sid = lax.axis_index("s")
        wid = cid * NS + sid
        slots = [(rowv0, tv0, c0v0, c1v0, c2v0), (rowv1, tv1, c0v1, c1v1, c2v1)]

        pltpu.sync_copy(zero_hbm, accl)

        offs = [0]
        for sz in sizes:
            offs.append(offs[-1] + sz)

        def load(c, sem):
            rv, tvv, a0, a1, a2 = slots[c % 2]
            sz = sizes[c]
            lbase = wid * epw + offs[c]
            return [
                pltpu.async_copy(row_hbm.at[pl.ds(e0 + lbase, sz)],
                                 rv.at[pl.ds(0, sz)], sem),
                pltpu.async_copy(t_hbm.at[pl.ds(lbase, sz)],
                                 tvv.at[pl.ds(0, sz)], sem),
                pltpu.async_copy(cd0_hbm.at[pl.ds(e0 + lbase, sz)],
                                 a0.at[pl.ds(0, sz)], sem),
                pltpu.async_copy(cd1_hbm.at[pl.ds(e0 + lbase, sz)],
                                 a1.at[pl.ds(0, sz)], sem),
                pltpu.async_copy(cd2_hbm.at[pl.ds(e0 + lbase, sz)],
                                 a2.at[pl.ds(0, sz)], sem),
            ]

        def body(sem4):
            pend = load(0, sem4)
            for c, sz in enumerate(sizes):
                for d in pend:
                    d.wait()
                if c + 1 < len(sizes):
                    pend = load(c + 1, sem4)
                rv_b, tv_b, a0_b, a1_b, a2_b = slots[c % 2]

                def grp(g, c2):
                    rv = rv_b[pl.ds(g * 16, 16)]
                    th = tv_b[pl.ds(g * 16, 16)]
                    f0 = rv * 3
                    for d, cdv in enumerate((a0_b, a1_b, a2_b)):
                        cd_d = cdv[pl.ds(g * 16, 16)]
                        plsc.addupdate_scatter(accl, [f0 + d], cd_d * th)
                    return c2

                lax.fori_loop(0, sz // 16, grp, 0)

        pl.run_scoped(body, pltpu.SemaphoreType.DMA)

        pltpu.sync_copy(accl, stage.at[sid])
        plsc.subcore_barrier()
        pltpu.sync_copy(stage.at[:, pl.ds(sid * sl, sl)], buf2)

        def colsum_rows(jr, carry):
            for jc in range(8):
                j = jr * 8 + jc
                acc16 = buf2[0, pl.ds(j * 16, 16)]
                for r in range(1, NS):
                    acc16 = acc16 + buf2[r, pl.ds(j * 16, 16)]
                res[jr, pl.ds(jc * 16, 16)] = acc16
            return carry

        lax.fori_loop(0, sl // 128, colsum_rows, 0)
        pltpu.sync_copy(res, out_hbm.at[cid, pl.ds(sid * (sl // 128),
                                                   sl // 128)])

    return k(t_loc, cd0, cd1, cd2, row1d, zeros1d)


def kernel(h, coord, edge_index, coord_diff, edge_attr, W1, b1, W2, b2, W3):
    n, hdim = h.shape
    e = edge_index.shape[1]

    w1a = W1[:, :hdim].astype(jnp.bfloat16)
    w1b = W1[:, hdim:2 * hdim].astype(jnp.bfloat16)
    w1c = W1[:, 2 * hdim:].astype(jnp.bfloat16)

    a_t, b_t = _node_ab(h, w1a, w1b)

    row = edge_index[0]
    col = edge_index[1]

    h0 = 204800
    gpad_a = 204800
    gpad_b = 122880
    epw_a = h0 // NW
    epw_b = (e - h0) // NW

    ea_t = jnp.pad(edge_attr, ((0, EPAD - e), (0, 0))).T
    b1r = b1.reshape(1, -1)
    b2r = b2.reshape(1, -1)
    w2c = W2.astype(jnp.bfloat16)
    w3c = W3.astype(jnp.bfloat16)
    zeros1d = jnp.zeros((ACC,), dtype=jnp.float32)
    cd0, cd1, cd2 = coord_diff[:, 0], coord_diff[:, 1], coord_diff[:, 2]

    g_a = _gather_sum(a_t, b_t, row, col, 0, epw_a, gpad_a)
    g_b = _gather_sum(a_t, b_t, row, col, h0, epw_b, gpad_b)

    t_a = _edge_mlp(g_a, ea_t, 0, w1c, b1r, w2c, b2r, w3c)
    t_b = _edge_mlp(g_b, ea_t, h0 // BE, w1c, b1r, w2c, b2r, w3c)

    pa = _segment_scatter(t_a, cd0, cd1, cd2, row, zeros1d, 0, epw_a)
    pb = _segment_scatter(t_b, cd0, cd1, cd2, row, zeros1d, h0, epw_b)

    agg = ((pa[0] + pa[1]) + (pb[0] + pb[1])).reshape(-1)[:3 * n].reshape(n, 3)
    return coord + agg
